# baseline (speedup 1.0000x reference)
import functools

import jax
import jax.numpy as jnp
from jax import lax
from jax.experimental import pallas as pl
from jax.experimental.pallas import tpu as pltpu
from jax.experimental.pallas import tpu_sc as plsc

_L = 16
_CH = 128


def _physflat(prob):
    n, c = prob.shape
    a = prob.reshape(n // 128, 128, c // 8, 8)
    return a.transpose(2, 0, 3, 1).reshape(-1)


def _nll_body(n, flat_hbm, tgt_hbm, w_hbm, fin_hbm,
              tgt_v, idx_v, pv_v, wv_v, out_v, sem):
    sid = lax.axis_index("s")

    @pl.when(sid == 0)
    def _():
        pltpu.sync_copy(tgt_hbm, tgt_v)

        iota = lax.iota(jnp.int32, _L)
        for j in range(n // _L):
            t16 = tgt_v[pl.ds(j * _L, _L)]
            i16 = j * _L + iota
            idx_v[pl.ds(j * _L, _L)] = (
                (t16 >> 3) * (8 * n) + ((i16 >> 7) << 10)
                + ((t16 & 7) << 7) + (i16 & 127))

        cps = []
        for j in range(n // _CH):
            s = pl.ds(j * _CH, _CH)
            cps.append(pltpu.async_copy(
                flat_hbm.at[idx_v.at[s]], pv_v.at[s], sem))
            cps.append(pltpu.async_copy(
                w_hbm.at[tgt_v.at[s]], wv_v.at[s], sem))
        for cp in cps:
            cp.wait()

        acc = jnp.zeros((_L,), jnp.float32)
        for j in range(n // _L):
            s = pl.ds(j * _L, _L)
            acc = acc + pv_v[s] * wv_v[s]
        for sh in (8, 4, 2, 1):
            acc = acc + jnp.take_along_axis(
                acc, iota ^ sh, axis=0, mode="promise_in_bounds")
        out_v[...] = -acc
        pltpu.sync_copy(out_v, fin_hbm)


def kernel(prob, target, weight):
    n, c = prob.shape
    assert n % _CH == 0 and n % 128 == 0 and c % 8 == 0

    mesh = plsc.VectorSubcoreMesh(
        core_axis_name="c", subcore_axis_name="s", num_cores=1)
    body = functools.partial(_nll_body, n)
    run = pl.kernel(
        body,
        out_type=jax.ShapeDtypeStruct((_L,), jnp.float32),
        mesh=mesh,
        compiler_params=pltpu.CompilerParams(
            needs_layout_passes=False, skip_device_barrier=True),
        scratch_types=[
            pltpu.VMEM((n,), jnp.int32),
            pltpu.VMEM((n,), jnp.int32),
            pltpu.VMEM((n,), jnp.float32),
            pltpu.VMEM((n,), jnp.float32),
            pltpu.VMEM((_L,), jnp.float32),
            pltpu.SemaphoreType.DMA,
        ],
    )
    fin = run(_physflat(prob), target, weight)
    return fin[0]


# speedup vs baseline: 1.1174x; 1.1174x over previous
import functools

import jax
import jax.numpy as jnp
from jax import lax
from jax.experimental import pallas as pl
from jax.experimental.pallas import tpu as pltpu
from jax.experimental.pallas import tpu_sc as plsc

_L = 16
_NS = 16


def _physflat(prob):
    n, c = prob.shape
    a = prob.reshape(n // 128, 128, c // 8, 8)
    return a.transpose(2, 0, 3, 1).reshape(-1)


def _nll_body(n, per_w, flat_hbm, tgt_hbm, w_hbm, part_hbm, fin_hbm,
              tgt_v, idx_v, pv_v, wv_v, stage_v, red_v, out_v, sem):
    sid = lax.axis_index("s")
    base = sid * per_w

    pltpu.sync_copy(tgt_hbm.at[pl.ds(base, per_w)], tgt_v)

    iota = lax.iota(jnp.int32, _L)
    for j in range(per_w // _L):
        t16 = tgt_v[pl.ds(j * _L, _L)]
        i16 = base + j * _L + iota
        idx_v[pl.ds(j * _L, _L)] = (
            (t16 >> 3) * (8 * n) + ((i16 >> 7) << 10)
            + ((t16 & 7) << 7) + (i16 & 127))

    cp_p = pltpu.async_copy(flat_hbm.at[idx_v], pv_v, sem)
    cp_w = pltpu.async_copy(w_hbm.at[tgt_v], wv_v, sem)
    cp_p.wait()
    cp_w.wait()

    acc = jnp.zeros((_L,), jnp.float32)
    for j in range(per_w // _L):
        acc = acc + pv_v[pl.ds(j * _L, _L)] * wv_v[pl.ds(j * _L, _L)]
    stage_v[...] = acc

    pltpu.sync_copy(stage_v, part_hbm.at[sid])
    plsc.subcore_barrier()

    @pl.when(sid == 0)
    def _():
        pltpu.sync_copy(part_hbm, red_v)
        tot = jnp.zeros((_L,), jnp.float32)
        for i in range(_NS):
            tot = tot + red_v[i]
        for sh in (8, 4, 2, 1):
            tot = tot + jnp.take_along_axis(
                tot, iota ^ sh, axis=0, mode="promise_in_bounds")
        out_v[...] = -tot
        pltpu.sync_copy(out_v, fin_hbm)


def kernel(prob, target, weight):
    n, c = prob.shape
    per_w = n // _NS
    assert per_w % _L == 0 and per_w * _NS == n
    assert n % 128 == 0 and c % 8 == 0

    mesh = plsc.VectorSubcoreMesh(
        core_axis_name="c", subcore_axis_name="s", num_cores=1)
    body = functools.partial(_nll_body, n, per_w)
    run = pl.kernel(
        body,
        out_type=(jax.ShapeDtypeStruct((_NS, _L), jnp.float32),
                  jax.ShapeDtypeStruct((_L,), jnp.float32)),
        mesh=mesh,
        compiler_params=pltpu.CompilerParams(
            needs_layout_passes=False, skip_device_barrier=True,
            disable_bounds_checks=True, disable_semaphore_checks=True),
        scratch_types=[
            pltpu.VMEM((per_w,), jnp.int32),
            pltpu.VMEM((per_w,), jnp.int32),
            pltpu.VMEM((per_w,), jnp.float32),
            pltpu.VMEM((per_w,), jnp.float32),
            pltpu.VMEM((_L,), jnp.float32),
            pltpu.VMEM((_NS, _L), jnp.float32),
            pltpu.VMEM((_L,), jnp.float32),
            pltpu.SemaphoreType.DMA,
        ],
    )
    _, fin = run(_physflat(prob), target, weight)
    return fin[0]


# speedup vs baseline: 1.1189x; 1.0013x over previous
import functools

import jax
import jax.numpy as jnp
from jax import lax
from jax.experimental import pallas as pl
from jax.experimental.pallas import tpu as pltpu
from jax.experimental.pallas import tpu_sc as plsc

_L = 16
_NS = 16


def _physflat(prob):
    n, c = prob.shape
    a = prob.reshape(n // 128, 128, c // 8, 8)
    return a.transpose(2, 0, 3, 1).reshape(-1)


def _nll_body(n, per_w, flat_hbm, tgt_hbm, w_hbm, part_hbm, fin_hbm,
              tgt_v, idx_v, pv_v, wv_v, stage_v, red_v, out_v, sem):
    sid = lax.axis_index("s")
    base = sid * per_w

    pltpu.sync_copy(tgt_hbm.at[pl.ds(base, per_w)], tgt_v)

    cp_w = pltpu.async_copy(w_hbm.at[tgt_v], wv_v, sem)

    iota = lax.iota(jnp.int32, _L)
    for j in range(per_w // _L):
        t16 = tgt_v[pl.ds(j * _L, _L)]
        i16 = base + j * _L + iota
        idx_v[pl.ds(j * _L, _L)] = (
            (t16 >> 3) * (8 * n) + ((i16 >> 7) << 10)
            + ((t16 & 7) << 7) + (i16 & 127))

    cp_p = pltpu.async_copy(flat_hbm.at[idx_v], pv_v, sem)
    cp_w.wait()
    cp_p.wait()

    acc = jnp.zeros((_L,), jnp.float32)
    for j in range(per_w // _L):
        acc = acc + pv_v[pl.ds(j * _L, _L)] * wv_v[pl.ds(j * _L, _L)]
    stage_v[...] = acc

    pltpu.sync_copy(stage_v, part_hbm.at[sid])
    plsc.subcore_barrier()

    @pl.when(sid == 0)
    def _():
        pltpu.sync_copy(part_hbm, red_v)
        tot = jnp.zeros((_L,), jnp.float32)
        for i in range(_NS):
            tot = tot + red_v[i]
        for sh in (8, 4, 2, 1):
            tot = tot + jnp.take_along_axis(
                tot, iota ^ sh, axis=0, mode="promise_in_bounds")
        out_v[...] = -tot
        pltpu.sync_copy(out_v, fin_hbm)


def kernel(prob, target, weight):
    n, c = prob.shape
    per_w = n // _NS
    assert per_w % _L == 0 and per_w * _NS == n
    assert n % 128 == 0 and c % 8 == 0

    mesh = plsc.VectorSubcoreMesh(
        core_axis_name="c", subcore_axis_name="s", num_cores=1)
    body = functools.partial(_nll_body, n, per_w)
    run = pl.kernel(
        body,
        out_type=(jax.ShapeDtypeStruct((_NS, _L), jnp.float32),
                  jax.ShapeDtypeStruct((_L,), jnp.float32)),
        mesh=mesh,
        compiler_params=pltpu.CompilerParams(
            needs_layout_passes=False, skip_device_barrier=True,
            disable_bounds_checks=True, disable_semaphore_checks=True),
        scratch_types=[
            pltpu.VMEM((per_w,), jnp.int32),
            pltpu.VMEM((per_w,), jnp.int32),
            pltpu.VMEM((per_w,), jnp.float32),
            pltpu.VMEM((per_w,), jnp.float32),
            pltpu.VMEM((_L,), jnp.float32),
            pltpu.VMEM((_NS, _L), jnp.float32),
            pltpu.VMEM((_L,), jnp.float32),
            pltpu.SemaphoreType.DMA,
        ],
    )
    _, fin = run(_physflat(prob), target, weight)
    return fin[0]


# speedup vs baseline: 1.1568x; 1.0338x over previous
import functools

import jax
import jax.numpy as jnp
from jax import lax
from jax.experimental import pallas as pl
from jax.experimental.pallas import tpu as pltpu
from jax.experimental.pallas import tpu_sc as plsc

_L = 16
_NS = 16


def _physflat(prob):
    n, c = prob.shape
    a = prob.reshape(n // 128, 128, c // 8, 8)
    return a.transpose(2, 0, 3, 1).reshape(-1)


def _nll_body(n, per_w, flat_hbm, tgt_hbm, w_hbm, fin_hbm,
              tgt_v, idx_v, pv_v, wv_v, stage_v, zero_v, red_v, out_v,
              idx0_v, acc_sh, sem):
    sid = lax.axis_index("s")
    base = sid * per_w

    pltpu.sync_copy(tgt_hbm.at[pl.ds(base, per_w)], tgt_v)

    cp_w = pltpu.async_copy(w_hbm.at[tgt_v], wv_v, sem)

    iota = lax.iota(jnp.int32, _L)
    idx0_v[...] = iota
    for j in range(per_w // _L):
        t16 = tgt_v[pl.ds(j * _L, _L)]
        i16 = base + j * _L + iota
        idx_v[pl.ds(j * _L, _L)] = (
            (t16 >> 3) * (8 * n) + ((i16 >> 7) << 10)
            + ((t16 & 7) << 7) + (i16 & 127))

    cp_p = pltpu.async_copy(flat_hbm.at[idx_v], pv_v, sem)

    @pl.when(sid == 0)
    def _():
        zero_v[...] = jnp.zeros((_L,), jnp.float32)
        pltpu.sync_copy(zero_v, acc_sh)

    cp_w.wait()
    cp_p.wait()

    acc = jnp.zeros((_L,), jnp.float32)
    for j in range(per_w // _L):
        acc = acc + pv_v[pl.ds(j * _L, _L)] * wv_v[pl.ds(j * _L, _L)]
    stage_v[...] = acc

    plsc.subcore_barrier()
    pltpu.sync_copy(stage_v, acc_sh.at[idx0_v], add=True)
    plsc.subcore_barrier()

    @pl.when(sid == 0)
    def _():
        pltpu.sync_copy(acc_sh, red_v)
        tot = red_v[...]
        for sh in (8, 4, 2, 1):
            tot = tot + jnp.take_along_axis(
                tot, iota ^ sh, axis=0, mode="promise_in_bounds")
        out_v[...] = -tot
        pltpu.sync_copy(out_v, fin_hbm)


def kernel(prob, target, weight):
    n, c = prob.shape
    per_w = n // _NS
    assert per_w % _L == 0 and per_w * _NS == n
    assert n % 128 == 0 and c % 8 == 0

    mesh = plsc.VectorSubcoreMesh(
        core_axis_name="c", subcore_axis_name="s", num_cores=1)
    body = functools.partial(_nll_body, n, per_w)
    run = pl.kernel(
        body,
        out_type=jax.ShapeDtypeStruct((_L,), jnp.float32),
        mesh=mesh,
        compiler_params=pltpu.CompilerParams(
            needs_layout_passes=False, skip_device_barrier=True,
            disable_bounds_checks=True, disable_semaphore_checks=True),
        scratch_types=[
            pltpu.VMEM((per_w,), jnp.int32),
            pltpu.VMEM((per_w,), jnp.int32),
            pltpu.VMEM((per_w,), jnp.float32),
            pltpu.VMEM((per_w,), jnp.float32),
            pltpu.VMEM((_L,), jnp.float32),
            pltpu.VMEM((_L,), jnp.float32),
            pltpu.VMEM((_L,), jnp.float32),
            pltpu.VMEM((_L,), jnp.float32),
            pltpu.VMEM((_L,), jnp.int32),
            pltpu.VMEM_SHARED((_L,), jnp.float32),
            pltpu.SemaphoreType.DMA,
        ],
    )
    fin = run(_physflat(prob), target, weight)
    return fin[0]
